# baseline (speedup 1.0000x reference)
import functools

import jax
import jax.numpy as jnp
import numpy as np
from jax import lax
from jax.experimental import pallas as pl
from jax.experimental.pallas import tpu as pltpu
from jax.experimental.pallas import tpu_sc as plsc

N = 50000
E = 800000
B = 512
DIM = 32
DXD = 78
NC = 2
NS = 16
K = 128

_CAP = 53760
_EBLK = _CAP // K
_CH = 28
_PBLK = -(-N // (NS * K))
_PBLK += _PBLK % 2
_PPAD = NS * _PBLK * K
_STR = 3128

_R = 2 * N // 4
_BR = 5000
_BN = 5000

_mesh = plsc.VectorSubcoreMesh(core_axis_name="c", subcore_axis_name="s")
_sc_params = pltpu.CompilerParams(use_tc_tiling_on_sc=False)


def _bdot(a, b):
    return jnp.dot(a.astype(jnp.bfloat16), b, preferred_element_type=jnp.float32)


def _make_sc_aggregate(width):
    @functools.partial(
        pl.kernel,
        out_type=jax.ShapeDtypeStruct((NC, N, width), jnp.float32),
        mesh=_mesh,
        compiler_params=_sc_params,
        scratch_types=[
            pltpu.VMEM_SHARED((N + 8, width), jnp.float32),
            pltpu.VMEM((_CH, K), jnp.int32),
            pltpu.VMEM((_CH, K), jnp.int32),
            pltpu.VMEM((4, K, width), jnp.float32),
            [pltpu.SemaphoreType.DMA] * 4,
        ],
    )
    def _sc_aggregate(h_hbm, src_hbm, dst_hbm, out_hbm, acc, srcv, dstv, rows,
                      sems):
        c = lax.axis_index("c")
        s = lax.axis_index("s")
        off = pl.multiple_of(jnp.minimum(s * _STR, N - _STR), 8)
        pltpu.sync_copy(
            h_hbm.at[pl.ds(c * N + off, _STR)], acc.at[pl.ds(off, _STR)]
        )
        plsc.subcore_barrier()

        @pl.loop(0, _EBLK // _CH)
        def _(j):
            pltpu.sync_copy(src_hbm.at[c, s, pl.ds(j * _CH, _CH)], srcv)
            pltpu.sync_copy(dst_hbm.at[c, s, pl.ds(j * _CH, _CH)], dstv)

            @pl.loop(0, _CH, step=4)
            def _(i):
                cps = [
                    pltpu.async_copy(h_hbm.at[srcv.at[i + b]], rows.at[b],
                                     sems[b])
                    for b in range(4)
                ]
                for b in range(4):
                    cps[b].wait()
                    pltpu.sync_copy(rows.at[b], acc.at[dstv.at[i + b]],
                                    add=True)

        plsc.subcore_barrier()
        pltpu.sync_copy(acc.at[pl.ds(off, _STR)], out_hbm.at[c, pl.ds(off, _STR)])

    return _sc_aggregate


_sc_agg32 = _make_sc_aggregate(DIM)
_sc_agg16 = _make_sc_aggregate(16)


@functools.partial(
    pl.kernel,
    out_type=jax.ShapeDtypeStruct((NC, B, DIM), jnp.float32),
    mesh=_mesh,
    compiler_params=_sc_params,
    scratch_types=[
        pltpu.VMEM_SHARED((B + 8, DIM), jnp.float32),
        pltpu.VMEM((_PBLK, K), jnp.int32),
        pltpu.VMEM((_PBLK, K), jnp.int32),
        pltpu.VMEM((2, K, DIM), jnp.float32),
        [pltpu.SemaphoreType.DMA] * 2,
    ],
)
def _sc_pool(h_hbm, src_hbm, dst_hbm, zero_hbm, out_hbm, acc, srcv, dstv, rows,
             sems):
    c = lax.axis_index("c")
    s = lax.axis_index("s")
    pltpu.sync_copy(src_hbm.at[c, s], srcv)
    pltpu.sync_copy(dst_hbm.at[c, s], dstv)

    @pl.when(s == 0)
    def _():
        pltpu.sync_copy(zero_hbm, acc)

    plsc.subcore_barrier()

    @pl.loop(0, _PBLK, step=2)
    def _(i):
        cps = [
            pltpu.async_copy(h_hbm.at[srcv.at[i + b]], rows.at[b], sems[b])
            for b in range(2)
        ]
        for b in range(2):
            cps[b].wait()
            pltpu.sync_copy(rows.at[b], acc.at[dstv.at[i + b]], add=True)

    plsc.subcore_barrier()

    @pl.when(s == 0)
    def _():
        pltpu.sync_copy(acc.at[pl.ds(0, B)], out_hbm.at[c])


def _tc_l0_body(a_ref, b_ref, c_ref, w1_ref, w2_ref, b1_ref, b2_ref, g_ref,
                bt_ref, o_ref):
    hs = jnp.concatenate([a_ref[...], b_ref[...], c_ref[...]], axis=1)
    z = _bdot(hs, w1_ref[...]) + b1_ref[...]
    u = jnp.maximum(z, 0.0)
    t = jnp.maximum(_bdot(u, w2_ref[...]) + b2_ref[...], 0.0)
    o_ref[...] = t * g_ref[...] + bt_ref[...]


def _tc_l0(hs_a, hs_b, hs_c, w1, w2, b1, b2, g, bt):
    return pl.pallas_call(
        _tc_l0_body,
        grid=(2 * N // _BN,),
        in_specs=[
            pl.BlockSpec((_BN, DIM), lambda i: (i, 0)),
            pl.BlockSpec((_BN, DIM), lambda i: (i, 0)),
            pl.BlockSpec((_BN, 16), lambda i: (i, 0)),
            pl.BlockSpec((80, DIM), lambda i: (0, 0)),
            pl.BlockSpec((DIM, DIM), lambda i: (0, 0)),
            pl.BlockSpec((1, DIM), lambda i: (0, 0)),
            pl.BlockSpec((1, DIM), lambda i: (0, 0)),
            pl.BlockSpec((1, DIM), lambda i: (0, 0)),
            pl.BlockSpec((1, DIM), lambda i: (0, 0)),
        ],
        out_specs=pl.BlockSpec((_BN, DIM), lambda i: (i, 0)),
        out_shape=jax.ShapeDtypeStruct((2 * N, DIM), jnp.float32),
    )(hs_a, hs_b, hs_c, w1, w2, b1, b2, g, bt)


def _tc_mid_body(hs_ref, w1_ref, w2_ref, b1_ref, b2_ref, g_ref, bt_ref, o_ref):
    z = _bdot(hs_ref[...], w1_ref[...]) + b1_ref[...]
    u = jnp.maximum(z, 0.0)
    t = jnp.maximum(_bdot(u, w2_ref[...]) + b2_ref[...], 0.0)
    o_ref[...] = t * g_ref[...] + bt_ref[...]


def _tc_mid(hsp, w1p, w2p, b1p, b2p, gp, btp):
    return pl.pallas_call(
        _tc_mid_body,
        grid=(_R // _BR,),
        in_specs=[
            pl.BlockSpec((_BR, 128), lambda i: (i, 0)),
            pl.BlockSpec((128, 128), lambda i: (0, 0)),
            pl.BlockSpec((128, 128), lambda i: (0, 0)),
            pl.BlockSpec((1, 128), lambda i: (0, 0)),
            pl.BlockSpec((1, 128), lambda i: (0, 0)),
            pl.BlockSpec((1, 128), lambda i: (0, 0)),
            pl.BlockSpec((1, 128), lambda i: (0, 0)),
        ],
        out_specs=pl.BlockSpec((_BR, 128), lambda i: (i, 0)),
        out_shape=jax.ShapeDtypeStruct((_R, 128), jnp.float32),
    )(hsp, w1p, w2p, b1p, b2p, gp, btp)


def _tc_head_body(g_ref, cell_ref, wf_ref, bf_ref, wr1_ref, br1_ref, wr2_ref,
                  br2_ref, wr3_ref, br3_ref, wf1_ref, bf1_ref, wf2_ref,
                  bf2_ref, wo_ref, bo_ref, o_ref):
    v = jnp.maximum(_bdot(g_ref[...], wf_ref[...]) + bf_ref[...], 0.0)
    cellp = cell_ref[...]
    nrm = jnp.sqrt(jnp.sum(cellp * cellp, axis=1, keepdims=True))
    cn = cellp / jnp.maximum(nrm, 1e-12)
    c1 = jnp.maximum(_bdot(cn, wr1_ref[...]) + br1_ref[...], 0.0)
    c2 = jnp.maximum(_bdot(c1, wr2_ref[...]) + br2_ref[...], 0.0)
    c3 = _bdot(c2, wr3_ref[...]) + br3_ref[...]
    xc = jnp.concatenate([v[:B], v[B:], c3], axis=1)
    f1 = jnp.maximum(_bdot(xc, wf1_ref[...]) + bf1_ref[...], 0.0)
    f2 = jnp.maximum(_bdot(f1, wf2_ref[...]) + bf2_ref[...], 0.0)
    o_ref[...] = _bdot(f2, wo_ref[...]) + bo_ref[...]


def _tc_head(g2, cellp, args):
    return pl.pallas_call(
        _tc_head_body,
        out_shape=jax.ShapeDtypeStruct((B, 128), jnp.float32),
    )(g2, cellp, *args)


def _prep_edges(ei, c):
    src, dst = ei[0], ei[1]
    bucket = dst // (N // NS)
    order = jnp.argsort(bucket, stable=True)
    srcs, dsts, buckets = src[order], dst[order], bucket[order]
    counts = jnp.bincount(buckets, length=NS)
    starts = jnp.concatenate([jnp.zeros((1,), jnp.int32),
                              jnp.cumsum(counts)[:-1].astype(jnp.int32)])
    pos = buckets * _CAP + (jnp.arange(E, dtype=jnp.int32) - starts[buckets])
    srcb = jnp.full((NS * _CAP,), c * N, jnp.int32).at[pos].set(srcs + c * N)
    dstb = jnp.full((NS * _CAP,), N, jnp.int32).at[pos].set(dsts)
    return srcb.reshape(NS, _EBLK, K), dstb.reshape(NS, _EBLK, K)


def _prep_pool(batch, c):
    pad = _PPAD - N
    src = jnp.concatenate(
        [jnp.arange(N, dtype=jnp.int32) + c * N, jnp.full((pad,), c * N, jnp.int32)]
    )
    dst = jnp.concatenate([batch, jnp.full((pad,), B, jnp.int32)])
    return src.reshape(NS, _PBLK, K), dst.reshape(NS, _PBLK, K)


def _kron4(w):
    return jnp.kron(jnp.eye(4, dtype=jnp.float32), w)


def _tile4(v):
    return jnp.tile(v, 4).reshape(1, 128)


def kernel(x1, edge_index1, batch1, x2, edge_index2, batch2, cell, params):
    gin = params["gin"]
    inv = np.float32(1.0) / np.float32(np.sqrt(np.float32(1.0 + 1e-5)))
    bf16 = jnp.bfloat16

    s1, d1 = _prep_edges(edge_index1, 0)
    s2, d2 = _prep_edges(edge_index2, 1)
    src_all = jnp.stack([s1, s2])
    dst_all = jnp.stack([d1, d2])
    ps1, pd1 = _prep_pool(batch1, 0)
    ps2, pd2 = _prep_pool(batch2, 1)
    psrc = jnp.stack([ps1, ps2])
    pdst = jnp.stack([pd1, pd2])
    zeros = jnp.zeros((B + 8, DIM), jnp.float32)

    x = jnp.concatenate([x1, x2], axis=0)
    xa = x[:, 0:32]
    xb = x[:, 32:64]
    xc = jnp.pad(x[:, 64:78], ((0, 0), (0, 2)))

    hs_a = _sc_agg32(xa, src_all, dst_all).reshape(2 * N, DIM)
    hs_b = _sc_agg32(xb, src_all, dst_all).reshape(2 * N, DIM)
    hs_c = _sc_agg16(xc, src_all, dst_all).reshape(2 * N, 16)
    lp = gin[0]
    w10 = jnp.pad(lp["W1"], ((0, 2), (0, 0))).astype(bf16)
    h = _tc_l0(hs_a, hs_b, hs_c, w10, lp["W2"].astype(bf16),
               lp["b1"].reshape(1, -1), lp["b2"].reshape(1, -1),
               (lp["gamma"] * inv).reshape(1, -1), lp["beta"].reshape(1, -1))

    for l in range(1, 5):
        lp = gin[l]
        hs = _sc_agg32(h, src_all, dst_all)
        h = _tc_mid(hs.reshape(_R, 128),
                    _kron4(lp["W1"]).astype(bf16), _kron4(lp["W2"]).astype(bf16),
                    _tile4(lp["b1"]), _tile4(lp["b2"]),
                    _tile4(lp["gamma"] * inv), _tile4(lp["beta"]))
        h = h.reshape(2 * N, DIM)

    g = _sc_pool(h, psrc, pdst, zeros)

    wf, bf = params["fc_xd"]
    wr1, br1 = params["red1"]
    wr2, br2 = params["red2"]
    wr3, br3 = params["red3"]
    wf1, bf1 = params["fc1"]
    wf2, bf2 = params["fc2"]
    wo, bo = params["out"]
    cellp = jnp.pad(cell, ((0, 0), (0, 1024 - cell.shape[1])))
    wr1p = jnp.pad(wr1, ((0, 1024 - wr1.shape[0]), (0, 0)))
    wop = jnp.pad(wo, ((0, 0), (0, 126)))
    bop = jnp.pad(bo, ((0, 126),)).reshape(1, 128)
    args = (wf.astype(bf16), bf.reshape(1, -1),
            wr1p.astype(bf16), br1.reshape(1, -1),
            wr2.astype(bf16), br2.reshape(1, -1),
            wr3.astype(bf16), br3.reshape(1, -1),
            wf1.astype(bf16), bf1.reshape(1, -1),
            wf2.astype(bf16), bf2.reshape(1, -1),
            wop.astype(bf16), bop)
    out = _tc_head(g.reshape(2 * B, DIM), cellp, args)
    return out[:, :2]


# speedup vs baseline: 8.3129x; 8.3129x over previous
import functools

import jax
import jax.numpy as jnp
import numpy as np
from jax import lax
from jax.experimental import pallas as pl
from jax.experimental.pallas import tpu as pltpu
from jax.experimental.pallas import tpu_sc as plsc

N = 50000
E = 800000
B = 512
DIM = 32
DXD = 78
NC = 2
NS = 16
K = 128

_EBLK = -(-E // (NS * K))
_EBLK += _EBLK % 2
_CH = 28
_EPAD = NS * _EBLK * K
_PBLK = -(-N // (NS * K))
_PBLK += _PBLK % 2
_PPAD = NS * _PBLK * K
_STR = 3128

_R = 2 * N // 4
_BR = 5000
_BN = 5000

_mesh = plsc.VectorSubcoreMesh(core_axis_name="c", subcore_axis_name="s")
_sc_params = pltpu.CompilerParams(use_tc_tiling_on_sc=False)


def _bdot(a, b):
    return jnp.dot(a.astype(jnp.bfloat16), b, preferred_element_type=jnp.float32)


def _make_sc_aggregate(width):
    @functools.partial(
        pl.kernel,
        out_type=jax.ShapeDtypeStruct((NC, N, width), jnp.float32),
        mesh=_mesh,
        compiler_params=_sc_params,
        scratch_types=[
            pltpu.VMEM_SHARED((N + 8, width), jnp.float32),
            pltpu.VMEM((_CH, K), jnp.int32),
            pltpu.VMEM((_CH, K), jnp.int32),
            pltpu.VMEM((4, K, width), jnp.float32),
            [pltpu.SemaphoreType.DMA] * 4,
        ],
    )
    def _sc_aggregate(h_hbm, zero_hbm, src_hbm, dst_hbm, out_hbm, acc, srcv,
                      dstv, rows, sems):
        c = lax.axis_index("c")
        s = lax.axis_index("s")
        off = pl.multiple_of(jnp.minimum(s * _STR, N - _STR), 8)
        pltpu.sync_copy(zero_hbm, acc.at[pl.ds(off, _STR)])
        plsc.subcore_barrier()

        @pl.loop(0, _EBLK // _CH)
        def _(j):
            pltpu.sync_copy(src_hbm.at[c, s, pl.ds(j * _CH, _CH)], srcv)
            pltpu.sync_copy(dst_hbm.at[c, s, pl.ds(j * _CH, _CH)], dstv)

            @pl.loop(0, _CH, step=4)
            def _(i):
                cps = [
                    pltpu.async_copy(h_hbm.at[srcv.at[i + b]], rows.at[b],
                                     sems[b])
                    for b in range(4)
                ]
                for b in range(4):
                    cps[b].wait()
                    pltpu.sync_copy(rows.at[b], acc.at[dstv.at[i + b]],
                                    add=True)

        plsc.subcore_barrier()
        pltpu.sync_copy(acc.at[pl.ds(off, _STR)], out_hbm.at[c, pl.ds(off, _STR)])

    return _sc_aggregate


_sc_agg32 = _make_sc_aggregate(DIM)
_sc_agg16 = _make_sc_aggregate(16)


@functools.partial(
    pl.kernel,
    out_type=jax.ShapeDtypeStruct((NC, B, DIM), jnp.float32),
    mesh=_mesh,
    compiler_params=_sc_params,
    scratch_types=[
        pltpu.VMEM_SHARED((B + 8, DIM), jnp.float32),
        pltpu.VMEM((_PBLK, K), jnp.int32),
        pltpu.VMEM((_PBLK, K), jnp.int32),
        pltpu.VMEM((2, K, DIM), jnp.float32),
        [pltpu.SemaphoreType.DMA] * 2,
    ],
)
def _sc_pool(h_hbm, src_hbm, dst_hbm, zero_hbm, out_hbm, acc, srcv, dstv, rows,
             sems):
    c = lax.axis_index("c")
    s = lax.axis_index("s")
    pltpu.sync_copy(src_hbm.at[c, s], srcv)
    pltpu.sync_copy(dst_hbm.at[c, s], dstv)

    @pl.when(s == 0)
    def _():
        pltpu.sync_copy(zero_hbm, acc)

    plsc.subcore_barrier()

    @pl.loop(0, _PBLK, step=2)
    def _(i):
        cps = [
            pltpu.async_copy(h_hbm.at[srcv.at[i + b]], rows.at[b], sems[b])
            for b in range(2)
        ]
        for b in range(2):
            cps[b].wait()
            pltpu.sync_copy(rows.at[b], acc.at[dstv.at[i + b]], add=True)

    plsc.subcore_barrier()

    @pl.when(s == 0)
    def _():
        pltpu.sync_copy(acc.at[pl.ds(0, B)], out_hbm.at[c])


def _tc_l0_body(a_ref, b_ref, c_ref, xa_ref, xb_ref, xc_ref, w1_ref, w2_ref,
                b1_ref, b2_ref, g_ref, bt_ref, o_ref):
    hs = jnp.concatenate([xa_ref[...] + a_ref[...], xb_ref[...] + b_ref[...],
                          xc_ref[...] + c_ref[...]], axis=1)
    z = _bdot(hs, w1_ref[...]) + b1_ref[...]
    u = jnp.maximum(z, 0.0)
    t = jnp.maximum(_bdot(u, w2_ref[...]) + b2_ref[...], 0.0)
    o_ref[...] = t * g_ref[...] + bt_ref[...]


def _tc_l0(hs_a, hs_b, hs_c, xa, xb, xc, w1, w2, b1, b2, g, bt):
    return pl.pallas_call(
        _tc_l0_body,
        grid=(2 * N // _BN,),
        in_specs=[
            pl.BlockSpec((_BN, DIM), lambda i: (i, 0)),
            pl.BlockSpec((_BN, DIM), lambda i: (i, 0)),
            pl.BlockSpec((_BN, 16), lambda i: (i, 0)),
            pl.BlockSpec((_BN, DIM), lambda i: (i, 0)),
            pl.BlockSpec((_BN, DIM), lambda i: (i, 0)),
            pl.BlockSpec((_BN, 16), lambda i: (i, 0)),
            pl.BlockSpec((80, DIM), lambda i: (0, 0)),
            pl.BlockSpec((DIM, DIM), lambda i: (0, 0)),
            pl.BlockSpec((1, DIM), lambda i: (0, 0)),
            pl.BlockSpec((1, DIM), lambda i: (0, 0)),
            pl.BlockSpec((1, DIM), lambda i: (0, 0)),
            pl.BlockSpec((1, DIM), lambda i: (0, 0)),
        ],
        out_specs=pl.BlockSpec((_BN, DIM), lambda i: (i, 0)),
        out_shape=jax.ShapeDtypeStruct((2 * N, DIM), jnp.float32),
    )(hs_a, hs_b, hs_c, xa, xb, xc, w1, w2, b1, b2, g, bt)


def _tc_mid_body(hs_ref, h_ref, w1_ref, w2_ref, b1_ref, b2_ref, g_ref, bt_ref,
                 o_ref):
    z = _bdot(h_ref[...] + hs_ref[...], w1_ref[...]) + b1_ref[...]
    u = jnp.maximum(z, 0.0)
    t = jnp.maximum(_bdot(u, w2_ref[...]) + b2_ref[...], 0.0)
    o_ref[...] = t * g_ref[...] + bt_ref[...]


def _tc_mid(hsp, hp, w1p, w2p, b1p, b2p, gp, btp):
    return pl.pallas_call(
        _tc_mid_body,
        grid=(_R // _BR,),
        in_specs=[
            pl.BlockSpec((_BR, 128), lambda i: (i, 0)),
            pl.BlockSpec((_BR, 128), lambda i: (i, 0)),
            pl.BlockSpec((128, 128), lambda i: (0, 0)),
            pl.BlockSpec((128, 128), lambda i: (0, 0)),
            pl.BlockSpec((1, 128), lambda i: (0, 0)),
            pl.BlockSpec((1, 128), lambda i: (0, 0)),
            pl.BlockSpec((1, 128), lambda i: (0, 0)),
            pl.BlockSpec((1, 128), lambda i: (0, 0)),
        ],
        out_specs=pl.BlockSpec((_BR, 128), lambda i: (i, 0)),
        out_shape=jax.ShapeDtypeStruct((_R, 128), jnp.float32),
    )(hsp, hp, w1p, w2p, b1p, b2p, gp, btp)


def _tc_head_body(g_ref, cell_ref, wf_ref, bf_ref, wr1_ref, br1_ref, wr2_ref,
                  br2_ref, wr3_ref, br3_ref, wf1_ref, bf1_ref, wf2_ref,
                  bf2_ref, wo_ref, bo_ref, o_ref):
    v = jnp.maximum(_bdot(g_ref[...], wf_ref[...]) + bf_ref[...], 0.0)
    cellp = cell_ref[...]
    nrm = jnp.sqrt(jnp.sum(cellp * cellp, axis=1, keepdims=True))
    cn = cellp / jnp.maximum(nrm, 1e-12)
    c1 = jnp.maximum(_bdot(cn, wr1_ref[...]) + br1_ref[...], 0.0)
    c2 = jnp.maximum(_bdot(c1, wr2_ref[...]) + br2_ref[...], 0.0)
    c3 = _bdot(c2, wr3_ref[...]) + br3_ref[...]
    xc = jnp.concatenate([v[:B], v[B:], c3], axis=1)
    f1 = jnp.maximum(_bdot(xc, wf1_ref[...]) + bf1_ref[...], 0.0)
    f2 = jnp.maximum(_bdot(f1, wf2_ref[...]) + bf2_ref[...], 0.0)
    o_ref[...] = _bdot(f2, wo_ref[...]) + bo_ref[...]


def _tc_head(g2, cellp, args):
    return pl.pallas_call(
        _tc_head_body,
        out_shape=jax.ShapeDtypeStruct((B, 128), jnp.float32),
    )(g2, cellp, *args)


def _prep_edges(ei, c):
    pad = _EPAD - E
    src = jnp.concatenate([ei[0] + c * N, jnp.full((pad,), c * N, jnp.int32)])
    dst = jnp.concatenate([ei[1], jnp.full((pad,), N, jnp.int32)])
    return src.reshape(NS, _EBLK, K), dst.reshape(NS, _EBLK, K)


def _prep_pool(batch, c):
    pad = _PPAD - N
    src = jnp.concatenate(
        [jnp.arange(N, dtype=jnp.int32) + c * N, jnp.full((pad,), c * N, jnp.int32)]
    )
    dst = jnp.concatenate([batch, jnp.full((pad,), B, jnp.int32)])
    return src.reshape(NS, _PBLK, K), dst.reshape(NS, _PBLK, K)


def _kron4(w):
    return jnp.kron(jnp.eye(4, dtype=jnp.float32), w)


def _tile4(v):
    return jnp.tile(v, 4).reshape(1, 128)


def kernel(x1, edge_index1, batch1, x2, edge_index2, batch2, cell, params):
    gin = params["gin"]
    inv = np.float32(1.0) / np.float32(np.sqrt(np.float32(1.0 + 1e-5)))
    bf16 = jnp.bfloat16

    s1, d1 = _prep_edges(edge_index1, 0)
    s2, d2 = _prep_edges(edge_index2, 1)
    src_all = jnp.stack([s1, s2])
    dst_all = jnp.stack([d1, d2])
    ps1, pd1 = _prep_pool(batch1, 0)
    ps2, pd2 = _prep_pool(batch2, 1)
    psrc = jnp.stack([ps1, ps2])
    pdst = jnp.stack([pd1, pd2])
    zeros = jnp.zeros((B + 8, DIM), jnp.float32)

    x = jnp.concatenate([x1, x2], axis=0)
    xa = x[:, 0:32]
    xb = x[:, 32:64]
    xc = jnp.pad(x[:, 64:78], ((0, 0), (0, 2)))

    z32 = jnp.zeros((_STR, DIM), jnp.float32)
    z16 = jnp.zeros((_STR, 16), jnp.float32)

    hs_a = _sc_agg32(xa, z32, src_all, dst_all).reshape(2 * N, DIM)
    hs_b = _sc_agg32(xb, z32, src_all, dst_all).reshape(2 * N, DIM)
    hs_c = _sc_agg16(xc, z16, src_all, dst_all).reshape(2 * N, 16)
    lp = gin[0]
    w10 = jnp.pad(lp["W1"], ((0, 2), (0, 0))).astype(bf16)
    h = _tc_l0(hs_a, hs_b, hs_c, xa, xb, xc, w10, lp["W2"].astype(bf16),
               lp["b1"].reshape(1, -1), lp["b2"].reshape(1, -1),
               (lp["gamma"] * inv).reshape(1, -1), lp["beta"].reshape(1, -1))

    for l in range(1, 5):
        lp = gin[l]
        hs = _sc_agg32(h, z32, src_all, dst_all)
        h = _tc_mid(hs.reshape(_R, 128), h.reshape(_R, 128),
                    _kron4(lp["W1"]).astype(bf16), _kron4(lp["W2"]).astype(bf16),
                    _tile4(lp["b1"]), _tile4(lp["b2"]),
                    _tile4(lp["gamma"] * inv), _tile4(lp["beta"]))
        h = h.reshape(2 * N, DIM)

    g = _sc_pool(h, psrc, pdst, zeros)

    wf, bf = params["fc_xd"]
    wr1, br1 = params["red1"]
    wr2, br2 = params["red2"]
    wr3, br3 = params["red3"]
    wf1, bf1 = params["fc1"]
    wf2, bf2 = params["fc2"]
    wo, bo = params["out"]
    cellp = jnp.pad(cell, ((0, 0), (0, 1024 - cell.shape[1])))
    wr1p = jnp.pad(wr1, ((0, 1024 - wr1.shape[0]), (0, 0)))
    wop = jnp.pad(wo, ((0, 0), (0, 126)))
    bop = jnp.pad(bo, ((0, 126),)).reshape(1, 128)
    args = (wf.astype(bf16), bf.reshape(1, -1),
            wr1p.astype(bf16), br1.reshape(1, -1),
            wr2.astype(bf16), br2.reshape(1, -1),
            wr3.astype(bf16), br3.reshape(1, -1),
            wf1.astype(bf16), bf1.reshape(1, -1),
            wf2.astype(bf16), bf2.reshape(1, -1),
            wop.astype(bf16), bop)
    out = _tc_head(g.reshape(2 * B, DIM), cellp, args)
    return out[:, :2]
